# baseline (device time: 132785 ns/iter reference)
import jax
import jax.numpy as jnp
from jax import lax
from jax.experimental import pallas as pl
from jax.experimental.pallas import tpu as pltpu

N_DEV = 4


def kernel(x, Win0, Wout0, Win1, Wout1, Win2, Wout2):
    b, d = x.shape
    h_per = Win0.shape[1]

    def body(x_ref, win0_ref, wout0_ref, win1_ref, wout1_ref, win2_ref,
             wout2_ref, out_ref, comm_win, comm_wout,
             send_sems_w, recv_sems_w, send_sems_o, recv_sems_o):
        my = lax.axis_index("i")
        left = lax.rem(my + N_DEV - 1, N_DEV)
        right = lax.rem(my + 1, N_DEV)
        barrier_sem = pltpu.get_barrier_semaphore()

        def barrier():
            for nbr in (left, right):
                pl.semaphore_signal(
                    barrier_sem, inc=1,
                    device_id=(nbr,), device_id_type=pl.DeviceIdType.MESH,
                )
            pl.semaphore_wait(barrier_sem, 2)

        def layer_step(xv, w, o, acc):
            hm = jnp.maximum(
                jnp.dot(xv, w, preferred_element_type=jnp.float32), 0.0)
            part = jnp.dot(hm, o, preferred_element_type=jnp.float32)
            return part if acc is None else acc + part

        xv = x_ref[...]
        layers = ((win0_ref, wout0_ref), (win1_ref, wout1_ref),
                  (win2_ref, wout2_ref))
        for win_ref, wout_ref in layers:
            barrier()
            acc = None
            for h in range(N_DEV - 1):
                slot = h % 2
                src_w = win_ref if h == 0 else comm_win.at[(h - 1) % 2]
                src_o = wout_ref if h == 0 else comm_wout.at[(h - 1) % 2]
                rw = pltpu.make_async_remote_copy(
                    src_ref=src_w, dst_ref=comm_win.at[slot],
                    send_sem=send_sems_w.at[slot],
                    recv_sem=recv_sems_w.at[slot],
                    device_id=(right,), device_id_type=pl.DeviceIdType.MESH,
                )
                ro = pltpu.make_async_remote_copy(
                    src_ref=src_o, dst_ref=comm_wout.at[slot],
                    send_sem=send_sems_o.at[slot],
                    recv_sem=recv_sems_o.at[slot],
                    device_id=(right,), device_id_type=pl.DeviceIdType.MESH,
                )
                rw.start()
                ro.start()
                acc = layer_step(xv, src_w[...], src_o[...], acc)
                rw.wait()
                ro.wait()
                if h == 1:
                    barrier()
            acc = layer_step(xv, comm_win[0], comm_wout[0], acc)
            xv = acc
        out_ref[...] = xv

    return pl.pallas_call(
        body,
        out_shape=jax.ShapeDtypeStruct((b, d), jnp.float32),
        in_specs=[pl.BlockSpec(memory_space=pltpu.VMEM)] * 7,
        out_specs=pl.BlockSpec(memory_space=pltpu.VMEM),
        scratch_shapes=[
            pltpu.VMEM((2, d, h_per), jnp.float32),
            pltpu.VMEM((2, h_per, d), jnp.float32),
            pltpu.SemaphoreType.DMA((2,)),
            pltpu.SemaphoreType.DMA((2,)),
            pltpu.SemaphoreType.DMA((2,)),
            pltpu.SemaphoreType.DMA((2,)),
        ],
        compiler_params=pltpu.CompilerParams(collective_id=0),
    )(x, Win0, Wout0, Win1, Wout1, Win2, Wout2)


# device time: 68346 ns/iter; 1.9428x vs baseline; 1.9428x over previous
import jax
import jax.numpy as jnp
from jax import lax
from jax.experimental import pallas as pl
from jax.experimental.pallas import tpu as pltpu

N_DEV = 4
N_LAYERS = 3
N_HOPS = N_DEV - 1


def kernel(x, Win0, Wout0, Win1, Wout1, Win2, Wout2):
    b, d = x.shape
    h_per = Win0.shape[1]
    hh = h_per // 2

    def body(x_ref, win0_ref, wout0_ref, win1_ref, wout1_ref, win2_ref,
             wout2_ref, out_ref, pack_r, pack_l, ring_r, ring_l,
             ss_r, rs_r, ss_l, rs_l):
        my = lax.axis_index("i")
        left = lax.rem(my + N_DEV - 1, N_DEV)
        right = lax.rem(my + 1, N_DEV)

        layers = ((win0_ref, wout0_ref), (win1_ref, wout1_ref),
                  (win2_ref, wout2_ref))

        for l, (win_ref, wout_ref) in enumerate(layers):
            pack_r[l, 0] = win_ref[:, 0:hh]
            pack_r[l, 1] = wout_ref[0:hh, :]
            pack_l[l, 0] = win_ref[:, hh:h_per]
            pack_l[l, 1] = wout_ref[hh:h_per, :]

        def make(l, h, ring, ss, rs, pack, tgt):
            src = pack.at[l] if h == 0 else ring.at[l, h - 1]
            return pltpu.make_async_remote_copy(
                src_ref=src, dst_ref=ring.at[l, h],
                send_sem=ss.at[l, h], recv_sem=rs.at[l, h],
                device_id=(tgt,), device_id_type=pl.DeviceIdType.MESH,
            )

        rr = {}
        rl = {}
        for l in range(N_LAYERS):
            for h in range(N_HOPS):
                rr[(l, h)] = make(l, h, ring_r, ss_r, rs_r, pack_r, right)
                rl[(l, h)] = make(l, h, ring_l, ss_l, rs_l, pack_l, left)

        barrier_sem = pltpu.get_barrier_semaphore()
        for nbr in (left, right):
            pl.semaphore_signal(
                barrier_sem, inc=1,
                device_id=(nbr,), device_id_type=pl.DeviceIdType.MESH,
            )
        pl.semaphore_wait(barrier_sem, 2)

        def start(l, h):
            rr[(l, h)].start()
            rl[(l, h)].start()

        def wait_recv(l, h):
            rr[(l, h)].wait_recv()
            rl[(l, h)].wait_recv()

        def half_step(xv, ring, l, h):
            hm = jnp.maximum(
                jnp.dot(xv, ring[l, h, 0],
                        preferred_element_type=jnp.float32), 0.0)
            return jnp.dot(hm, ring[l, h, 1],
                           preferred_element_type=jnp.float32)

        def both_halves(xv, l, h):
            return half_step(xv, ring_r, l, h) + half_step(xv, ring_l, l, h)

        def own_step(xv, l):
            win_ref, wout_ref = layers[l]
            hm = jnp.maximum(
                jnp.dot(xv, win_ref[...],
                        preferred_element_type=jnp.float32), 0.0)
            return jnp.dot(hm, wout_ref[...],
                           preferred_element_type=jnp.float32)

        xv = x_ref[...]

        start(0, 0)
        start(1, 0)
        acc = own_step(xv, 0)
        wait_recv(0, 0)
        start(0, 1)
        start(2, 0)
        acc = acc + both_halves(xv, 0, 0)
        wait_recv(0, 1)
        start(0, 2)
        acc = acc + both_halves(xv, 0, 1)
        wait_recv(1, 0)
        start(1, 1)
        wait_recv(0, 2)
        acc = acc + both_halves(xv, 0, 2)
        xv = acc

        acc = own_step(xv, 1) + both_halves(xv, 1, 0)
        wait_recv(1, 1)
        start(1, 2)
        acc = acc + both_halves(xv, 1, 1)
        wait_recv(2, 0)
        start(2, 1)
        wait_recv(1, 2)
        acc = acc + both_halves(xv, 1, 2)
        xv = acc

        acc = own_step(xv, 2) + both_halves(xv, 2, 0)
        wait_recv(2, 1)
        start(2, 2)
        acc = acc + both_halves(xv, 2, 1)
        wait_recv(2, 2)
        acc = acc + both_halves(xv, 2, 2)
        out_ref[...] = acc

        for l in range(N_LAYERS):
            for h in range(N_HOPS):
                rr[(l, h)].wait_send()
                rl[(l, h)].wait_send()

    return pl.pallas_call(
        body,
        out_shape=jax.ShapeDtypeStruct((b, d), jnp.float32),
        in_specs=[pl.BlockSpec(memory_space=pltpu.VMEM)] * 7,
        out_specs=pl.BlockSpec(memory_space=pltpu.VMEM),
        scratch_shapes=[
            pltpu.VMEM((N_LAYERS, 2, d, hh), jnp.float32),
            pltpu.VMEM((N_LAYERS, 2, d, hh), jnp.float32),
            pltpu.VMEM((N_LAYERS, N_HOPS, 2, d, hh), jnp.float32),
            pltpu.VMEM((N_LAYERS, N_HOPS, 2, d, hh), jnp.float32),
            pltpu.SemaphoreType.DMA((N_LAYERS, N_HOPS)),
            pltpu.SemaphoreType.DMA((N_LAYERS, N_HOPS)),
            pltpu.SemaphoreType.DMA((N_LAYERS, N_HOPS)),
            pltpu.SemaphoreType.DMA((N_LAYERS, N_HOPS)),
        ],
        compiler_params=pltpu.CompilerParams(collective_id=0),
    )(x, Win0, Wout0, Win1, Wout1, Win2, Wout2)


# device time: 39616 ns/iter; 3.3518x vs baseline; 1.7252x over previous
import jax
import jax.numpy as jnp
from jax import lax
from jax.experimental import pallas as pl
from jax.experimental.pallas import tpu as pltpu

N_DEV = 4
N_LAYERS = 3
N_HOPS = N_DEV - 1


def kernel(x, Win0, Wout0, Win1, Wout1, Win2, Wout2):
    b, d = x.shape
    h_per = Win0.shape[1]
    hh = h_per // 2

    def body(x_ref, win0_ref, wout0_ref, win1_ref, wout1_ref, win2_ref,
             wout2_ref, out_ref, pack_r, pack_l, ring_r, ring_l,
             ss_r, rs_r, ss_l, rs_l):
        my = lax.axis_index("i")
        left = lax.rem(my + N_DEV - 1, N_DEV)
        right = lax.rem(my + 1, N_DEV)

        layers = ((win0_ref, wout0_ref), (win1_ref, wout1_ref),
                  (win2_ref, wout2_ref))

        def make(l, h, ring, ss, rs, pack, tgt):
            src = pack.at[l] if h == 0 else ring.at[l, h - 1]
            return pltpu.make_async_remote_copy(
                src_ref=src, dst_ref=ring.at[l, h],
                send_sem=ss.at[l, h], recv_sem=rs.at[l, h],
                device_id=(tgt,), device_id_type=pl.DeviceIdType.MESH,
            )

        rr = {}
        rl = {}
        for l in range(N_LAYERS):
            for h in range(N_HOPS):
                rr[(l, h)] = make(l, h, ring_r, ss_r, rs_r, pack_r, right)
                rl[(l, h)] = make(l, h, ring_l, ss_l, rs_l, pack_l, left)

        barrier_sem = pltpu.get_barrier_semaphore()
        for nbr in (left, right):
            pl.semaphore_signal(
                barrier_sem, inc=1,
                device_id=(nbr,), device_id_type=pl.DeviceIdType.MESH,
            )
        pl.semaphore_wait(barrier_sem, 2)

        def start(l, h):
            rr[(l, h)].start()
            rl[(l, h)].start()

        def wait_recv(l, h):
            rr[(l, h)].wait_recv()
            rl[(l, h)].wait_recv()

        for l, (win_ref, wout_ref) in enumerate(layers):
            pack_r[l, 0] = win_ref[:, 0:hh].astype(jnp.bfloat16)
            pack_r[l, 1] = wout_ref[0:hh, :].astype(jnp.bfloat16)
            pack_l[l, 0] = win_ref[:, hh:h_per].astype(jnp.bfloat16)
            pack_l[l, 1] = wout_ref[hh:h_per, :].astype(jnp.bfloat16)
            start(l, 0)

        def half_step(xb, ring, l, h):
            hm = jnp.maximum(
                jnp.dot(xb, ring[l, h, 0],
                        preferred_element_type=jnp.float32), 0.0)
            return jnp.dot(hm.astype(jnp.bfloat16), ring[l, h, 1],
                           preferred_element_type=jnp.float32)

        def both_halves(xb, l, h):
            return half_step(xb, ring_r, l, h) + half_step(xb, ring_l, l, h)

        def own_step(xb, l):
            win_ref, wout_ref = layers[l]
            hm = jnp.maximum(
                jnp.dot(xb, win_ref[...].astype(jnp.bfloat16),
                        preferred_element_type=jnp.float32), 0.0)
            return jnp.dot(hm.astype(jnp.bfloat16),
                           wout_ref[...].astype(jnp.bfloat16),
                           preferred_element_type=jnp.float32)

        xb = x_ref[...].astype(jnp.bfloat16)
        acc0 = own_step(xb, 0)
        wait_recv(0, 0)
        start(0, 1)
        acc0 = acc0 + both_halves(xb, 0, 0)
        wait_recv(1, 0)
        start(1, 1)
        wait_recv(2, 0)
        start(2, 1)
        wait_recv(0, 1)
        start(0, 2)
        acc0 = acc0 + both_halves(xb, 0, 1)
        wait_recv(1, 1)
        start(1, 2)
        wait_recv(2, 1)
        start(2, 2)
        wait_recv(0, 2)
        acc0 = acc0 + both_halves(xb, 0, 2)

        xb = acc0.astype(jnp.bfloat16)
        acc1 = own_step(xb, 1) + both_halves(xb, 1, 0) + both_halves(xb, 1, 1)
        wait_recv(1, 2)
        acc1 = acc1 + both_halves(xb, 1, 2)

        xb = acc1.astype(jnp.bfloat16)
        acc2 = own_step(xb, 2) + both_halves(xb, 2, 0) + both_halves(xb, 2, 1)
        wait_recv(2, 2)
        acc2 = acc2 + both_halves(xb, 2, 2)
        out_ref[...] = acc2

        for l in range(N_LAYERS):
            for h in range(N_HOPS):
                rr[(l, h)].wait_send()
                rl[(l, h)].wait_send()

    return pl.pallas_call(
        body,
        out_shape=jax.ShapeDtypeStruct((b, d), jnp.float32),
        in_specs=[pl.BlockSpec(memory_space=pltpu.VMEM)] * 7,
        out_specs=pl.BlockSpec(memory_space=pltpu.VMEM),
        scratch_shapes=[
            pltpu.VMEM((N_LAYERS, 2, d, hh), jnp.bfloat16),
            pltpu.VMEM((N_LAYERS, 2, d, hh), jnp.bfloat16),
            pltpu.VMEM((N_LAYERS, N_HOPS, 2, d, hh), jnp.bfloat16),
            pltpu.VMEM((N_LAYERS, N_HOPS, 2, d, hh), jnp.bfloat16),
            pltpu.SemaphoreType.DMA((N_LAYERS, N_HOPS)),
            pltpu.SemaphoreType.DMA((N_LAYERS, N_HOPS)),
            pltpu.SemaphoreType.DMA((N_LAYERS, N_HOPS)),
            pltpu.SemaphoreType.DMA((N_LAYERS, N_HOPS)),
        ],
        compiler_params=pltpu.CompilerParams(collective_id=0),
    )(x, Win0, Wout0, Win1, Wout1, Win2, Wout2)
